# baseline (speedup 1.0000x reference)
import functools

import jax
import jax.numpy as jnp
from jax import lax
from jax.experimental import pallas as pl
from jax.experimental.pallas import tpu as pltpu
from jax.experimental.pallas import tpu_sc as plsc

N_COLS_K = 26
VOCAB_K = 100000
EMB_K = 8
BATCH_K = 16384

_NC = 2
_NS = 16
_NW = _NC * _NS
_ROWS_TOTAL = BATCH_K * N_COLS_K
_ROWS_PER_W = _ROWS_TOTAL // _NW


def _gather_body(table_hbm, idx_hbm, out_hbm, idx_v, rows_v, sem):
    wid = lax.axis_index("s") * _NC + lax.axis_index("c")
    base = wid * _ROWS_PER_W
    pltpu.sync_copy(idx_hbm.at[pl.ds(base, _ROWS_PER_W)], idx_v)
    pltpu.async_copy(table_hbm.at[idx_v], rows_v, sem).wait()
    pltpu.sync_copy(rows_v, out_hbm.at[pl.ds(base, _ROWS_PER_W)])


_sc_gather = pl.kernel(
    _gather_body,
    out_type=jax.ShapeDtypeStruct((_ROWS_TOTAL, EMB_K), jnp.float32),
    mesh=plsc.VectorSubcoreMesh(core_axis_name="c", subcore_axis_name="s"),
    scratch_types=[
        pltpu.VMEM((_ROWS_PER_W,), jnp.int32),
        pltpu.VMEM((_ROWS_PER_W, EMB_K), jnp.float32),
        pltpu.SemaphoreType.DMA,
    ],
    compiler_params=pltpu.CompilerParams(use_tc_tiling_on_sc=False),
)


def kernel(x_batch, tables):
    flat_idx = (
        x_batch + jnp.arange(N_COLS_K, dtype=jnp.int32)[None, :] * VOCAB_K
    ).reshape(-1)
    table_flat = tables.reshape(N_COLS_K * VOCAB_K, EMB_K)
    rows = _sc_gather(table_flat, flat_idx)
    return rows.reshape(BATCH_K, N_COLS_K * EMB_K)


# speedup vs baseline: 1.0002x; 1.0002x over previous
import jax
import jax.numpy as jnp
from jax import lax
from jax.experimental import pallas as pl
from jax.experimental.pallas import tpu as pltpu
from jax.experimental.pallas import tpu_sc as plsc

N_COLS_K = 26
VOCAB_K = 100000
EMB_K = 8
BATCH_K = 16384

_NC = 2
_NS = 16
_NW = _NC * _NS

_ROWS_TOTAL = BATCH_K * N_COLS_K
_ROWS_PER_W = _ROWS_TOTAL // _NW


def _gather_body(table_hbm, idx_hbm, out_hbm, idx_v, rows_v, sem):
    wid = lax.axis_index("s") * _NC + lax.axis_index("c")
    base = wid * _ROWS_PER_W
    pltpu.sync_copy(idx_hbm.at[pl.ds(base, _ROWS_PER_W)], idx_v)
    pltpu.async_copy(table_hbm.at[idx_v], rows_v, sem).wait()
    pltpu.sync_copy(rows_v, out_hbm.at[pl.ds(base, _ROWS_PER_W)])


_sc_gather = pl.kernel(
    _gather_body,
    out_type=jax.ShapeDtypeStruct((_ROWS_TOTAL, EMB_K), jnp.float32),
    mesh=plsc.VectorSubcoreMesh(core_axis_name="c", subcore_axis_name="s"),
    scratch_types=[
        pltpu.VMEM((_ROWS_PER_W,), jnp.int32),
        pltpu.VMEM((_ROWS_PER_W, EMB_K), jnp.float32),
        pltpu.SemaphoreType.DMA,
    ],
    compiler_params=pltpu.CompilerParams(use_tc_tiling_on_sc=False),
)


def kernel(x_batch, tables):
    t128 = lax.optimization_barrier(
        tables.reshape(N_COLS_K * VOCAB_K // 16, 16 * EMB_K)
    )
    table_flat = t128.reshape(N_COLS_K * VOCAB_K, EMB_K)
    flat_idx = (
        x_batch + jnp.arange(N_COLS_K, dtype=jnp.int32)[None, :] * VOCAB_K
    ).reshape(-1)
    rows = _sc_gather(table_flat, flat_idx)
    return rows.reshape(BATCH_K, N_COLS_K * EMB_K)
